# baseline (speedup 1.0000x reference)
import jax
import jax.numpy as jnp
from jax import lax
from jax.experimental import pallas as pl
from jax.experimental.pallas import tpu as pltpu

D = 2048
KSEL = 256
B = 8
NPIX = B * 32 * 32
SIGMA = 0.1

A = 128
G = 16

BLK = 512


def _build_et(mu, noise, extra, et_ref):
    z = mu + SIGMA * (noise + 0.25 * extra)
    gate = jnp.clip(z + 0.5, 0.0, 1.0)

    bits = lax.bitcast_convert_type(gate, jnp.int32)
    bits = jnp.where(bits < 0, 0, bits)

    def bs_step(i, lo):
        cand = lo | (1 << (30 - i))
        cnt = jnp.sum((bits >= cand).astype(jnp.int32))
        return jnp.where(cnt >= KSEL, cand, lo)

    thresh = lax.fori_loop(0, 31, bs_step, jnp.int32(0))
    maskf = (bits >= thresh).astype(jnp.float32)

    ia = lax.broadcasted_iota(jnp.int32, (A, A), 0)
    ja = lax.broadcasted_iota(jnp.int32, (A, A), 1)
    lower = (ja <= ia).astype(jnp.float32)
    colcs = jnp.dot(lower, maskf, preferred_element_type=jnp.float32)
    coltot = colcs[A - 1:A, :]
    ig = lax.broadcasted_iota(jnp.int32, (G, G), 0)
    jg = lax.broadcasted_iota(jnp.int32, (G, G), 1)
    strict = (ig < jg).astype(jnp.float32)
    prefix = jnp.dot(coltot, strict, preferred_element_type=jnp.float32)
    ranks = (colcs + prefix) * maskf

    jlane = lax.broadcasted_iota(jnp.int32, (A, KSEL), 1).astype(jnp.float32)
    for g in range(G):
        rank_col = jnp.broadcast_to(ranks[:, g:g + 1], (A, KSEL))
        gate_col = jnp.broadcast_to(gate[:, g:g + 1], (A, KSEL))
        hit = rank_col == jlane + 1.0
        et_ref[pl.ds(g * A, A), :] = jnp.where(
            hit, gate_col, 0.0).astype(jnp.bfloat16)


def _fused_body(mu_ref, noise_ref, extra_ref, x_ref, out_ref, et_ref):
    @pl.when(pl.program_id(0) == 0)
    def _():
        _build_et(mu_ref[...], noise_ref[...], extra_ref[...], et_ref)

    out_ref[...] = lax.dot_general(
        x_ref[...].astype(jnp.bfloat16), et_ref[...],
        (((1,), (0,)), ((), ())), preferred_element_type=jnp.float32)


def _fused(mu2, noise2, extra2, x2):
    return pl.pallas_call(
        _fused_body,
        grid=(NPIX // BLK,),
        in_specs=[
            pl.BlockSpec((A, G), lambda i: (0, 0)),
            pl.BlockSpec((A, G), lambda i: (0, 0)),
            pl.BlockSpec((A, G), lambda i: (0, 0)),
            pl.BlockSpec((BLK, D), lambda i: (i, 0)),
        ],
        out_specs=pl.BlockSpec((BLK, KSEL), lambda i: (i, 0)),
        out_shape=jax.ShapeDtypeStruct((NPIX, KSEL), jnp.float32),
        scratch_shapes=[pltpu.VMEM((D, KSEL), jnp.bfloat16)],
    )(mu2, noise2, extra2, x2)


def kernel(x, mu, noise, extra_noise):
    x2 = x.reshape(B, D, 32, 32).transpose(0, 2, 3, 1).reshape(NPIX, D)
    grid = lambda a: a.reshape(G, A).T
    out2 = _fused(grid(mu), grid(noise), grid(extra_noise), x2)
    return out2.reshape(B, 32, 32, KSEL).transpose(0, 3, 1, 2)[:, None]


# speedup vs baseline: 1.0976x; 1.0976x over previous
import jax
import jax.numpy as jnp
from jax import lax
from jax.experimental import pallas as pl
from jax.experimental.pallas import tpu as pltpu

D = 2048
KSEL = 256
B = 8
NPIX = B * 32 * 32
SIGMA = 0.1

A = 128
G = 16

BLK = 1024


def _build_et(mu, noise, extra, et_ref):
    z = mu + SIGMA * (noise + 0.25 * extra)
    gate = jnp.clip(z + 0.5, 0.0, 1.0)

    bits = lax.bitcast_convert_type(gate, jnp.int32)
    bits = jnp.where(bits < 0, 0, bits)

    def bs_step(i, lo):
        cand = lo | (1 << (30 - i))
        cnt = jnp.sum((bits >= cand).astype(jnp.int32))
        return jnp.where(cnt >= KSEL, cand, lo)

    thresh = lax.fori_loop(0, 31, bs_step, jnp.int32(0))
    maskf = (bits >= thresh).astype(jnp.float32)

    ia = lax.broadcasted_iota(jnp.int32, (A, A), 0)
    ja = lax.broadcasted_iota(jnp.int32, (A, A), 1)
    lower = (ja <= ia).astype(jnp.float32)
    colcs = jnp.dot(lower, maskf, preferred_element_type=jnp.float32)
    coltot = colcs[A - 1:A, :]
    ig = lax.broadcasted_iota(jnp.int32, (G, G), 0)
    jg = lax.broadcasted_iota(jnp.int32, (G, G), 1)
    strict = (ig < jg).astype(jnp.float32)
    prefix = jnp.dot(coltot, strict, preferred_element_type=jnp.float32)
    ranks = (colcs + prefix) * maskf

    jlane = lax.broadcasted_iota(jnp.int32, (A, KSEL), 1).astype(jnp.float32)
    for g in range(G):
        rank_col = jnp.broadcast_to(ranks[:, g:g + 1], (A, KSEL))
        gate_col = jnp.broadcast_to(gate[:, g:g + 1], (A, KSEL))
        hit = rank_col == jlane + 1.0
        et_ref[pl.ds(g * A, A), :] = jnp.where(
            hit, gate_col, 0.0).astype(jnp.bfloat16)


def _fused_body(mu_ref, noise_ref, extra_ref, x_ref, out_ref, et_ref):
    @pl.when(pl.program_id(0) == 0)
    def _():
        _build_et(mu_ref[...], noise_ref[...], extra_ref[...], et_ref)

    out_ref[...] = lax.dot_general(
        x_ref[...].astype(jnp.bfloat16), et_ref[...],
        (((1,), (0,)), ((), ())), preferred_element_type=jnp.float32)


def _fused(mu2, noise2, extra2, x2):
    return pl.pallas_call(
        _fused_body,
        grid=(NPIX // BLK,),
        in_specs=[
            pl.BlockSpec((A, G), lambda i: (0, 0)),
            pl.BlockSpec((A, G), lambda i: (0, 0)),
            pl.BlockSpec((A, G), lambda i: (0, 0)),
            pl.BlockSpec((BLK, D), lambda i: (i, 0)),
        ],
        out_specs=pl.BlockSpec((BLK, KSEL), lambda i: (i, 0)),
        out_shape=jax.ShapeDtypeStruct((NPIX, KSEL), jnp.float32),
        scratch_shapes=[pltpu.VMEM((D, KSEL), jnp.bfloat16)],
    )(mu2, noise2, extra2, x2)


def kernel(x, mu, noise, extra_noise):
    x2 = x.reshape(B, D, 32, 32).transpose(0, 2, 3, 1).reshape(NPIX, D)
    grid = lambda a: a.reshape(G, A).T
    out2 = _fused(grid(mu), grid(noise), grid(extra_noise), x2)
    return out2.reshape(B, 32, 32, KSEL).transpose(0, 3, 1, 2)[:, None]
